# baseline (speedup 1.0000x reference)
import functools

import jax
import jax.numpy as jnp
from jax import lax
from jax.experimental import pallas as pl
from jax.experimental.pallas import tpu as pltpu
from jax.experimental.pallas import tpu_sc as plsc

N_NODES = 10000
N_EDGES = 320000
D = 128
DH = 64

C = 128
NW = 32
NP = 10240
EP_CHUNKS = 2560
EP_TOT_CHUNKS = 2592
EP = EP_TOT_CHUNKS * C
CHUNKS_PER_W = EP_CHUNKS // NW
ZCH_PER_SUB = NP // C // 16
DEGW = 16
DUMMY_DST = N_NODES + 8


def _sc_body(with_deg, *refs):
    if with_deg:
        (ylo_hbm, yhi_hbm, src_hbm, dst_hbm,
         outlo_hbm, outhi_hbm, deg_hbm,
         src_i0, dst_i0, src_i1, dst_i1, rows0_v, rows1_v, zbuf_v, ones_v,
         accum_s, deg_s, sem0, sem1) = refs
    else:
        (ylo_hbm, yhi_hbm, src_hbm, dst_hbm,
         outlo_hbm, outhi_hbm,
         src_i0, dst_i0, src_i1, dst_i1, rows0_v, rows1_v, zbuf_v,
         accum_s, sem0, sem1) = refs

    c = lax.axis_index("c")
    s = lax.axis_index("s")
    wid = s * 2 + c

    def zrow(r, _):
        def zcol(k, _):
            zbuf_v[r, pl.ds(k * 16, 16)] = jnp.zeros((16,), jnp.float32)
            return 0
        return lax.fori_loop(0, DH // 16, zcol, 0)
    lax.fori_loop(0, C, zrow, 0)

    if with_deg:
        def zo(r, _):
            ones_v[r, :] = jnp.zeros((DEGW,), jnp.float32)
            return 0
        lax.fori_loop(0, C, zo, 0)

    def zero_accum():
        for j in range(ZCH_PER_SUB):
            pltpu.sync_copy(zbuf_v, accum_s.at[pl.ds((s + 16 * j) * C, C)])

    zero_accum()
    if with_deg:
        for j in range(ZCH_PER_SUB):
            pltpu.sync_copy(ones_v, deg_s.at[pl.ds((s + 16 * j) * C, C)])

        def so(r, _):
            ones_v[r, :] = jnp.full((DEGW,), 1.0, jnp.float32)
            return 0
        lax.fori_loop(0, C, so, 0)

    plsc.subcore_barrier()

    def edge_pass(y_ref, add_deg):
        def load_idx(k, si, di):
            base = (k * NW + wid) * C
            pltpu.sync_copy(src_hbm.at[pl.ds(base, C)], si)
            pltpu.sync_copy(dst_hbm.at[pl.ds(base, C)], di)

        load_idx(0, src_i0, dst_i0)
        pltpu.async_copy(y_ref.at[src_i0], rows0_v, sem0)

        @pl.loop(0, CHUNKS_PER_W, step=2)
        def chunk(j):
            load_idx(j + 1, src_i1, dst_i1)
            pltpu.async_copy(y_ref.at[src_i1], rows1_v, sem1)
            pltpu.make_async_copy(y_ref.at[src_i0], rows0_v, sem0).wait()
            pltpu.sync_copy(rows0_v, accum_s.at[dst_i0], add=True)
            if add_deg:
                pltpu.sync_copy(ones_v, deg_s.at[dst_i0], add=True)
            load_idx(j + 2, src_i0, dst_i0)
            pltpu.async_copy(y_ref.at[src_i0], rows0_v, sem0)
            pltpu.make_async_copy(y_ref.at[src_i1], rows1_v, sem1).wait()
            pltpu.sync_copy(rows1_v, accum_s.at[dst_i1], add=True)
            if add_deg:
                pltpu.sync_copy(ones_v, deg_s.at[dst_i1], add=True)

        pltpu.make_async_copy(y_ref.at[src_i0], rows0_v, sem0).wait()

    def writeback(dst_ref):
        for j in range(ZCH_PER_SUB):
            off = (s + 16 * j) * C
            pltpu.sync_copy(accum_s.at[pl.ds(off, C)],
                            dst_ref.at[c, pl.ds(off, C)])

    edge_pass(ylo_hbm, with_deg)
    plsc.subcore_barrier()
    writeback(outlo_hbm)
    if with_deg:
        for j in range(ZCH_PER_SUB):
            off = (s + 16 * j) * C
            pltpu.sync_copy(deg_s.at[pl.ds(off, C)],
                            deg_hbm.at[c, pl.ds(off, C)])
    zero_accum()
    plsc.subcore_barrier()

    edge_pass(yhi_hbm, False)
    plsc.subcore_barrier()
    writeback(outhi_hbm)


def _make_sc_aggregate(with_deg):
    mesh = plsc.VectorSubcoreMesh(core_axis_name="c", subcore_axis_name="s")
    out_type = [
        jax.ShapeDtypeStruct((2, NP, DH), jnp.float32),
        jax.ShapeDtypeStruct((2, NP, DH), jnp.float32),
    ]
    scratch = [
        pltpu.VMEM((C,), jnp.int32),
        pltpu.VMEM((C,), jnp.int32),
        pltpu.VMEM((C,), jnp.int32),
        pltpu.VMEM((C,), jnp.int32),
        pltpu.VMEM((C, DH), jnp.float32),
        pltpu.VMEM((C, DH), jnp.float32),
        pltpu.VMEM((C, DH), jnp.float32),
    ]
    if with_deg:
        out_type.append(jax.ShapeDtypeStruct((2, NP, DEGW), jnp.float32))
        scratch.append(pltpu.VMEM((C, DEGW), jnp.float32))
    scratch.append(pltpu.VMEM_SHARED((NP, DH), jnp.float32))
    if with_deg:
        scratch.append(pltpu.VMEM_SHARED((NP, DEGW), jnp.float32))
    scratch.append(pltpu.SemaphoreType.DMA)
    scratch.append(pltpu.SemaphoreType.DMA)

    return pl.kernel(
        functools.partial(_sc_body, with_deg),
        out_type=tuple(out_type),
        mesh=mesh,
        scratch_types=tuple(scratch),
        compiler_params=pltpu.CompilerParams(use_tc_tiling_on_sc=False),
    )


_sc_aggregate_deg = _make_sc_aggregate(True)
_sc_aggregate = _make_sc_aggregate(False)


R = 400
GRID = N_NODES // R


def _mm_body(x_ref, w_ref, lo_ref, hi_ref):
    y = jnp.dot(x_ref[...], w_ref[...], preferred_element_type=jnp.float32)
    lo_ref[...] = y[:, :DH]
    hi_ref[...] = y[:, DH:]


def _tc_matmul_split(x, w):
    return pl.pallas_call(
        _mm_body,
        grid=(GRID,),
        in_specs=[
            pl.BlockSpec((R, D), lambda i: (i, 0)),
            pl.BlockSpec((D, D), lambda i: (0, 0)),
        ],
        out_specs=[
            pl.BlockSpec((R, DH), lambda i: (i, 0)),
            pl.BlockSpec((R, DH), lambda i: (i, 0)),
        ],
        out_shape=[
            jax.ShapeDtypeStruct((N_NODES, DH), jnp.float32),
            jax.ShapeDtypeStruct((N_NODES, DH), jnp.float32),
        ],
    )(x, w)


def _mid_body(plo_ref, phi_ref, deg_ref, x_ref, wr0_ref, b0_ref, wl1_ref,
              h_ref, y1lo_ref, y1hi_ref):
    agg = jnp.concatenate(
        [plo_ref[0] + plo_ref[1], phi_ref[0] + phi_ref[1]], axis=1)
    d = deg_ref[0, :, 0:1] + deg_ref[1, :, 0:1]
    agg = agg / jnp.maximum(d, 1.0)
    h = agg + jnp.dot(x_ref[...], wr0_ref[...],
                      preferred_element_type=jnp.float32) + b0_ref[...]
    h = jnp.maximum(h, 0.0)
    h_ref[...] = h
    y1 = jnp.dot(h, wl1_ref[...], preferred_element_type=jnp.float32)
    y1lo_ref[...] = y1[:, :DH]
    y1hi_ref[...] = y1[:, DH:]


def _tc_mid(p0lo, p0hi, degp, x, wr0, b0, wl1):
    return pl.pallas_call(
        _mid_body,
        grid=(GRID,),
        in_specs=[
            pl.BlockSpec((2, R, DH), lambda i: (0, i, 0)),
            pl.BlockSpec((2, R, DH), lambda i: (0, i, 0)),
            pl.BlockSpec((2, R, DEGW), lambda i: (0, i, 0)),
            pl.BlockSpec((R, D), lambda i: (i, 0)),
            pl.BlockSpec((D, D), lambda i: (0, 0)),
            pl.BlockSpec((1, D), lambda i: (0, 0)),
            pl.BlockSpec((D, D), lambda i: (0, 0)),
        ],
        out_specs=[
            pl.BlockSpec((R, D), lambda i: (i, 0)),
            pl.BlockSpec((R, DH), lambda i: (i, 0)),
            pl.BlockSpec((R, DH), lambda i: (i, 0)),
        ],
        out_shape=[
            jax.ShapeDtypeStruct((N_NODES, D), jnp.float32),
            jax.ShapeDtypeStruct((N_NODES, DH), jnp.float32),
            jax.ShapeDtypeStruct((N_NODES, DH), jnp.float32),
        ],
    )(p0lo, p0hi, degp, x, wr0, b0, wl1)


def _fin_body(plo_ref, phi_ref, deg_ref, h_ref, wr1_ref, b1_ref, o_ref):
    agg = jnp.concatenate(
        [plo_ref[0] + plo_ref[1], phi_ref[0] + phi_ref[1]], axis=1)
    d = deg_ref[0, :, 0:1] + deg_ref[1, :, 0:1]
    agg = agg / jnp.maximum(d, 1.0)
    o_ref[...] = agg + jnp.dot(h_ref[...], wr1_ref[...],
                               preferred_element_type=jnp.float32) + b1_ref[...]


def _tc_final(p1lo, p1hi, degp, h, wr1, b1):
    return pl.pallas_call(
        _fin_body,
        grid=(GRID,),
        in_specs=[
            pl.BlockSpec((2, R, DH), lambda i: (0, i, 0)),
            pl.BlockSpec((2, R, DH), lambda i: (0, i, 0)),
            pl.BlockSpec((2, R, DEGW), lambda i: (0, i, 0)),
            pl.BlockSpec((R, D), lambda i: (i, 0)),
            pl.BlockSpec((D, D), lambda i: (0, 0)),
            pl.BlockSpec((1, D), lambda i: (0, 0)),
        ],
        out_specs=pl.BlockSpec((R, D), lambda i: (i, 0)),
        out_shape=jax.ShapeDtypeStruct((N_NODES, D), jnp.float32),
    )(p1lo, p1hi, degp, h, wr1, b1)


@jax.jit
def kernel(x, edge_index, Wl0, Wr0, b0, Wl1, Wr1, b1):
    ei = edge_index.astype(jnp.int32)
    pad = EP - N_EDGES
    src = jnp.concatenate([ei[0], jnp.zeros((pad,), jnp.int32)])
    dst = jnp.concatenate([ei[1], jnp.full((pad,), DUMMY_DST, jnp.int32)])
    b0r = b0.reshape(1, D)
    b1r = b1.reshape(1, D)

    y0lo, y0hi = _tc_matmul_split(x, Wl0)
    p0lo, p0hi, degp = _sc_aggregate_deg(y0lo, y0hi, src, dst)
    h, y1lo, y1hi = _tc_mid(p0lo, p0hi, degp, x, Wr0, b0r, Wl1)
    p1lo, p1hi = _sc_aggregate(y1lo, y1hi, src, dst)
    return _tc_final(p1lo, p1hi, degp, h, Wr1, b1r)


# speedup vs baseline: 1.7086x; 1.7086x over previous
import functools

import jax
import jax.numpy as jnp
from jax import lax
from jax.experimental import pallas as pl
from jax.experimental.pallas import tpu as pltpu
from jax.experimental.pallas import tpu_sc as plsc

N_NODES = 10000
N_EDGES = 320000
D = 128
DH = 64

C = 128
NW = 32
NP = 10240
EP_CHUNKS = 2528
EP = EP_CHUNKS * C
CHUNKS_PER_W = EP_CHUNKS // NW
ZCH_PER_SUB = NP // C // 16
DEGW = 16
DUMMY_DST = N_NODES + 8


def _sc_body(with_deg, *refs):
    if with_deg:
        (ylo_hbm, yhi_hbm, src_hbm, dst_hbm,
         outlo_hbm, outhi_hbm, deg_hbm,
         src_all, dst_all, rows_v, zbuf_v, ones_v,
         accum_s, deg_s, sem) = refs
    else:
        (ylo_hbm, yhi_hbm, src_hbm, dst_hbm,
         outlo_hbm, outhi_hbm,
         src_all, dst_all, rows_v, zbuf_v,
         accum_s, sem) = refs

    c = lax.axis_index("c")
    s = lax.axis_index("s")
    wid = s * 2 + c

    def zrow(r, _):
        def zcol(k, _):
            zbuf_v[r, pl.ds(k * 16, 16)] = jnp.zeros((16,), jnp.float32)
            return 0
        return lax.fori_loop(0, DH // 16, zcol, 0)
    lax.fori_loop(0, C, zrow, 0)

    if with_deg:
        def zo(r, _):
            ones_v[r, :] = jnp.zeros((DEGW,), jnp.float32)
            return 0
        lax.fori_loop(0, C, zo, 0)

    def zero_accum():
        for j in range(ZCH_PER_SUB):
            pltpu.sync_copy(zbuf_v, accum_s.at[pl.ds((s + 16 * j) * C, C)])

    pltpu.sync_copy(src_hbm.at[wid], src_all)
    pltpu.sync_copy(dst_hbm.at[wid], dst_all)

    zero_accum()
    if with_deg:
        for j in range(ZCH_PER_SUB):
            pltpu.sync_copy(ones_v, deg_s.at[pl.ds((s + 16 * j) * C, C)])

        def so(r, _):
            ones_v[r, :] = jnp.full((DEGW,), 1.0, jnp.float32)
            return 0
        lax.fori_loop(0, C, so, 0)

    plsc.subcore_barrier()

    def edge_pass(y_ref, add_deg):
        @pl.loop(0, CHUNKS_PER_W)
        def chunk(j):
            pltpu.async_copy(y_ref.at[src_all.at[j]], rows_v, sem).wait()
            pltpu.sync_copy(rows_v, accum_s.at[dst_all.at[j]], add=True)
            if add_deg:
                pltpu.sync_copy(ones_v, deg_s.at[dst_all.at[j]], add=True)

    def writeback(dst_ref):
        for j in range(ZCH_PER_SUB):
            off = (s + 16 * j) * C
            pltpu.sync_copy(accum_s.at[pl.ds(off, C)],
                            dst_ref.at[c, pl.ds(off, C)])

    edge_pass(ylo_hbm, with_deg)
    plsc.subcore_barrier()
    writeback(outlo_hbm)
    if with_deg:
        for j in range(ZCH_PER_SUB):
            off = (s + 16 * j) * C
            pltpu.sync_copy(deg_s.at[pl.ds(off, C)],
                            deg_hbm.at[c, pl.ds(off, C)])
    zero_accum()
    plsc.subcore_barrier()

    edge_pass(yhi_hbm, False)
    plsc.subcore_barrier()
    writeback(outhi_hbm)


def _make_sc_aggregate(with_deg):
    mesh = plsc.VectorSubcoreMesh(core_axis_name="c", subcore_axis_name="s")
    out_type = [
        jax.ShapeDtypeStruct((2, NP, DH), jnp.float32),
        jax.ShapeDtypeStruct((2, NP, DH), jnp.float32),
    ]
    scratch = [
        pltpu.VMEM((CHUNKS_PER_W, C), jnp.int32),
        pltpu.VMEM((CHUNKS_PER_W, C), jnp.int32),
        pltpu.VMEM((C, DH), jnp.float32),
        pltpu.VMEM((C, DH), jnp.float32),
    ]
    if with_deg:
        out_type.append(jax.ShapeDtypeStruct((2, NP, DEGW), jnp.float32))
        scratch.append(pltpu.VMEM((C, DEGW), jnp.float32))
    scratch.append(pltpu.VMEM_SHARED((NP, DH), jnp.float32))
    if with_deg:
        scratch.append(pltpu.VMEM_SHARED((NP, DEGW), jnp.float32))
    scratch.append(pltpu.SemaphoreType.DMA)

    return pl.kernel(
        functools.partial(_sc_body, with_deg),
        out_type=tuple(out_type),
        mesh=mesh,
        scratch_types=tuple(scratch),
        compiler_params=pltpu.CompilerParams(use_tc_tiling_on_sc=False),
    )


_sc_aggregate_deg = _make_sc_aggregate(True)
_sc_aggregate = _make_sc_aggregate(False)


R = 400
GRID = N_NODES // R


def _mm_body(x_ref, w_ref, lo_ref, hi_ref):
    y = jnp.dot(x_ref[...], w_ref[...], preferred_element_type=jnp.float32)
    lo_ref[...] = y[:, :DH]
    hi_ref[...] = y[:, DH:]


def _tc_matmul_split(x, w):
    return pl.pallas_call(
        _mm_body,
        grid=(GRID,),
        in_specs=[
            pl.BlockSpec((R, D), lambda i: (i, 0)),
            pl.BlockSpec((D, D), lambda i: (0, 0)),
        ],
        out_specs=[
            pl.BlockSpec((R, DH), lambda i: (i, 0)),
            pl.BlockSpec((R, DH), lambda i: (i, 0)),
        ],
        out_shape=[
            jax.ShapeDtypeStruct((N_NODES, DH), jnp.float32),
            jax.ShapeDtypeStruct((N_NODES, DH), jnp.float32),
        ],
    )(x, w)


def _mid_body(plo_ref, phi_ref, deg_ref, x_ref, wr0_ref, b0_ref, wl1_ref,
              h_ref, y1lo_ref, y1hi_ref):
    agg = jnp.concatenate(
        [plo_ref[0] + plo_ref[1], phi_ref[0] + phi_ref[1]], axis=1)
    d = deg_ref[0, :, 0:1] + deg_ref[1, :, 0:1]
    agg = agg / jnp.maximum(d, 1.0)
    h = agg + jnp.dot(x_ref[...], wr0_ref[...],
                      preferred_element_type=jnp.float32) + b0_ref[...]
    h = jnp.maximum(h, 0.0)
    h_ref[...] = h
    y1 = jnp.dot(h, wl1_ref[...], preferred_element_type=jnp.float32)
    y1lo_ref[...] = y1[:, :DH]
    y1hi_ref[...] = y1[:, DH:]


def _tc_mid(p0lo, p0hi, degp, x, wr0, b0, wl1):
    return pl.pallas_call(
        _mid_body,
        grid=(GRID,),
        in_specs=[
            pl.BlockSpec((2, R, DH), lambda i: (0, i, 0)),
            pl.BlockSpec((2, R, DH), lambda i: (0, i, 0)),
            pl.BlockSpec((2, R, DEGW), lambda i: (0, i, 0)),
            pl.BlockSpec((R, D), lambda i: (i, 0)),
            pl.BlockSpec((D, D), lambda i: (0, 0)),
            pl.BlockSpec((1, D), lambda i: (0, 0)),
            pl.BlockSpec((D, D), lambda i: (0, 0)),
        ],
        out_specs=[
            pl.BlockSpec((R, D), lambda i: (i, 0)),
            pl.BlockSpec((R, DH), lambda i: (i, 0)),
            pl.BlockSpec((R, DH), lambda i: (i, 0)),
        ],
        out_shape=[
            jax.ShapeDtypeStruct((N_NODES, D), jnp.float32),
            jax.ShapeDtypeStruct((N_NODES, DH), jnp.float32),
            jax.ShapeDtypeStruct((N_NODES, DH), jnp.float32),
        ],
    )(p0lo, p0hi, degp, x, wr0, b0, wl1)


def _fin_body(plo_ref, phi_ref, deg_ref, h_ref, wr1_ref, b1_ref, o_ref):
    agg = jnp.concatenate(
        [plo_ref[0] + plo_ref[1], phi_ref[0] + phi_ref[1]], axis=1)
    d = deg_ref[0, :, 0:1] + deg_ref[1, :, 0:1]
    agg = agg / jnp.maximum(d, 1.0)
    o_ref[...] = agg + jnp.dot(h_ref[...], wr1_ref[...],
                               preferred_element_type=jnp.float32) + b1_ref[...]


def _tc_final(p1lo, p1hi, degp, h, wr1, b1):
    return pl.pallas_call(
        _fin_body,
        grid=(GRID,),
        in_specs=[
            pl.BlockSpec((2, R, DH), lambda i: (0, i, 0)),
            pl.BlockSpec((2, R, DH), lambda i: (0, i, 0)),
            pl.BlockSpec((2, R, DEGW), lambda i: (0, i, 0)),
            pl.BlockSpec((R, D), lambda i: (i, 0)),
            pl.BlockSpec((D, D), lambda i: (0, 0)),
            pl.BlockSpec((1, D), lambda i: (0, 0)),
        ],
        out_specs=pl.BlockSpec((R, D), lambda i: (i, 0)),
        out_shape=jax.ShapeDtypeStruct((N_NODES, D), jnp.float32),
    )(p1lo, p1hi, degp, h, wr1, b1)


@jax.jit
def kernel(x, edge_index, Wl0, Wr0, b0, Wl1, Wr1, b1):
    ei = edge_index.astype(jnp.int32)
    pad = EP - N_EDGES
    src = jnp.concatenate([ei[0], jnp.zeros((pad,), jnp.int32)])
    dst = jnp.concatenate([ei[1], jnp.full((pad,), DUMMY_DST, jnp.int32)])
    src = src.reshape(CHUNKS_PER_W, NW, C).transpose(1, 0, 2)
    dst = dst.reshape(CHUNKS_PER_W, NW, C).transpose(1, 0, 2)
    b0r = b0.reshape(1, D)
    b1r = b1.reshape(1, D)

    y0lo, y0hi = _tc_matmul_split(x, Wl0)
    p0lo, p0hi, degp = _sc_aggregate_deg(y0lo, y0hi, src, dst)
    h, y1lo, y1hi = _tc_mid(p0lo, p0hi, degp, x, Wr0, b0r, Wl1)
    p1lo, p1hi = _sc_aggregate(y1lo, y1hi, src, dst)
    return _tc_final(p1lo, p1hi, degp, h, Wr1, b1r)


# speedup vs baseline: 2.1849x; 1.2788x over previous
import functools

import jax
import jax.numpy as jnp
from jax import lax
from jax.experimental import pallas as pl
from jax.experimental.pallas import tpu as pltpu
from jax.experimental.pallas import tpu_sc as plsc

N_NODES = 10000
N_EDGES = 320000
D = 128
DH = 64

C = 128
NW = 32
NP = 10240
EP_CHUNKS = 2528
EP = EP_CHUNKS * C
CHUNKS_PER_W = EP_CHUNKS // NW
ZCH_PER_SUB = NP // C // 16
DEGW = 16
DUMMY_DST = N_NODES + 8


def _sc_body(with_deg, *refs):
    if with_deg:
        (ylo_hbm, yhi_hbm, src_hbm, dst_hbm,
         outlo_hbm, outhi_hbm, deg_hbm,
         src_all, dst_all, rows_v, rows1_v, zbuf_v, ones_v,
         accum_s, deg_s, sem, sem1) = refs
    else:
        (ylo_hbm, yhi_hbm, src_hbm, dst_hbm,
         outlo_hbm, outhi_hbm,
         src_all, dst_all, rows_v, rows1_v, zbuf_v,
         accum_s, sem, sem1) = refs

    c = lax.axis_index("c")
    s = lax.axis_index("s")
    wid = s * 2 + c

    def zrow(r, _):
        def zcol(k, _):
            zbuf_v[r, pl.ds(k * 16, 16)] = jnp.zeros((16,), jnp.float32)
            return 0
        return lax.fori_loop(0, DH // 16, zcol, 0)
    lax.fori_loop(0, C, zrow, 0)

    if with_deg:
        def zo(r, _):
            ones_v[r, :] = jnp.zeros((DEGW,), jnp.float32)
            return 0
        lax.fori_loop(0, C, zo, 0)

    def zero_accum():
        for j in range(ZCH_PER_SUB):
            pltpu.sync_copy(zbuf_v, accum_s.at[pl.ds((s + 16 * j) * C, C)])

    pltpu.sync_copy(src_hbm.at[wid], src_all)
    pltpu.sync_copy(dst_hbm.at[wid], dst_all)

    zero_accum()
    if with_deg:
        for j in range(ZCH_PER_SUB):
            pltpu.sync_copy(ones_v, deg_s.at[pl.ds((s + 16 * j) * C, C)])

        def so(r, _):
            ones_v[r, :] = jnp.full((DEGW,), 1.0, jnp.float32)
            return 0
        lax.fori_loop(0, C, so, 0)

    plsc.subcore_barrier()

    def edge_pass(y_ref, add_deg):
        def scat(buf, j):
            pltpu.sync_copy(buf, accum_s.at[dst_all.at[j]], add=True)
            if add_deg:
                pltpu.sync_copy(ones_v, deg_s.at[dst_all.at[j]], add=True)

        pltpu.async_copy(y_ref.at[src_all.at[0]], rows_v, sem)

        @pl.loop(0, CHUNKS_PER_W - 1, step=2)
        def chunk(j):
            pltpu.async_copy(y_ref.at[src_all.at[j + 1]], rows1_v, sem1)
            pltpu.make_async_copy(y_ref.at[src_all.at[j]], rows_v, sem).wait()
            scat(rows_v, j)
            pltpu.async_copy(y_ref.at[src_all.at[j + 2]], rows_v, sem)
            pltpu.make_async_copy(
                y_ref.at[src_all.at[j + 1]], rows1_v, sem1).wait()
            scat(rows1_v, j + 1)

        pltpu.make_async_copy(
            y_ref.at[src_all.at[CHUNKS_PER_W - 1]], rows_v, sem).wait()
        scat(rows_v, CHUNKS_PER_W - 1)

    def writeback(dst_ref):
        for j in range(ZCH_PER_SUB):
            off = (s + 16 * j) * C
            pltpu.sync_copy(accum_s.at[pl.ds(off, C)],
                            dst_ref.at[c, pl.ds(off, C)])

    edge_pass(ylo_hbm, with_deg)
    plsc.subcore_barrier()
    writeback(outlo_hbm)
    if with_deg:
        for j in range(ZCH_PER_SUB):
            off = (s + 16 * j) * C
            pltpu.sync_copy(deg_s.at[pl.ds(off, C)],
                            deg_hbm.at[c, pl.ds(off, C)])
    zero_accum()
    plsc.subcore_barrier()

    edge_pass(yhi_hbm, False)
    plsc.subcore_barrier()
    writeback(outhi_hbm)


def _make_sc_aggregate(with_deg):
    mesh = plsc.VectorSubcoreMesh(core_axis_name="c", subcore_axis_name="s")
    out_type = [
        jax.ShapeDtypeStruct((2, NP, DH), jnp.float32),
        jax.ShapeDtypeStruct((2, NP, DH), jnp.float32),
    ]
    scratch = [
        pltpu.VMEM((CHUNKS_PER_W, C), jnp.int32),
        pltpu.VMEM((CHUNKS_PER_W, C), jnp.int32),
        pltpu.VMEM((C, DH), jnp.float32),
        pltpu.VMEM((C, DH), jnp.float32),
        pltpu.VMEM((C, DH), jnp.float32),
    ]
    if with_deg:
        out_type.append(jax.ShapeDtypeStruct((2, NP, DEGW), jnp.float32))
        scratch.append(pltpu.VMEM((C, DEGW), jnp.float32))
    scratch.append(pltpu.VMEM_SHARED((NP, DH), jnp.float32))
    if with_deg:
        scratch.append(pltpu.VMEM_SHARED((NP, DEGW), jnp.float32))
    scratch.append(pltpu.SemaphoreType.DMA)
    scratch.append(pltpu.SemaphoreType.DMA)

    return pl.kernel(
        functools.partial(_sc_body, with_deg),
        out_type=tuple(out_type),
        mesh=mesh,
        scratch_types=tuple(scratch),
        compiler_params=pltpu.CompilerParams(use_tc_tiling_on_sc=False),
    )


_sc_aggregate_deg = _make_sc_aggregate(True)
_sc_aggregate = _make_sc_aggregate(False)


R = 400
GRID = N_NODES // R


def _mm_body(x_ref, w_ref, lo_ref, hi_ref):
    y = jnp.dot(x_ref[...], w_ref[...], preferred_element_type=jnp.float32)
    lo_ref[...] = y[:, :DH]
    hi_ref[...] = y[:, DH:]


def _tc_matmul_split(x, w):
    return pl.pallas_call(
        _mm_body,
        grid=(GRID,),
        in_specs=[
            pl.BlockSpec((R, D), lambda i: (i, 0)),
            pl.BlockSpec((D, D), lambda i: (0, 0)),
        ],
        out_specs=[
            pl.BlockSpec((R, DH), lambda i: (i, 0)),
            pl.BlockSpec((R, DH), lambda i: (i, 0)),
        ],
        out_shape=[
            jax.ShapeDtypeStruct((N_NODES, DH), jnp.float32),
            jax.ShapeDtypeStruct((N_NODES, DH), jnp.float32),
        ],
    )(x, w)


def _mid_body(plo_ref, phi_ref, deg_ref, x_ref, wr0_ref, b0_ref, wl1_ref,
              h_ref, y1lo_ref, y1hi_ref):
    agg = jnp.concatenate(
        [plo_ref[0] + plo_ref[1], phi_ref[0] + phi_ref[1]], axis=1)
    d = deg_ref[0, :, 0:1] + deg_ref[1, :, 0:1]
    agg = agg / jnp.maximum(d, 1.0)
    h = agg + jnp.dot(x_ref[...], wr0_ref[...],
                      preferred_element_type=jnp.float32) + b0_ref[...]
    h = jnp.maximum(h, 0.0)
    h_ref[...] = h
    y1 = jnp.dot(h, wl1_ref[...], preferred_element_type=jnp.float32)
    y1lo_ref[...] = y1[:, :DH]
    y1hi_ref[...] = y1[:, DH:]


def _tc_mid(p0lo, p0hi, degp, x, wr0, b0, wl1):
    return pl.pallas_call(
        _mid_body,
        grid=(GRID,),
        in_specs=[
            pl.BlockSpec((2, R, DH), lambda i: (0, i, 0)),
            pl.BlockSpec((2, R, DH), lambda i: (0, i, 0)),
            pl.BlockSpec((2, R, DEGW), lambda i: (0, i, 0)),
            pl.BlockSpec((R, D), lambda i: (i, 0)),
            pl.BlockSpec((D, D), lambda i: (0, 0)),
            pl.BlockSpec((1, D), lambda i: (0, 0)),
            pl.BlockSpec((D, D), lambda i: (0, 0)),
        ],
        out_specs=[
            pl.BlockSpec((R, D), lambda i: (i, 0)),
            pl.BlockSpec((R, DH), lambda i: (i, 0)),
            pl.BlockSpec((R, DH), lambda i: (i, 0)),
        ],
        out_shape=[
            jax.ShapeDtypeStruct((N_NODES, D), jnp.float32),
            jax.ShapeDtypeStruct((N_NODES, DH), jnp.float32),
            jax.ShapeDtypeStruct((N_NODES, DH), jnp.float32),
        ],
    )(p0lo, p0hi, degp, x, wr0, b0, wl1)


def _fin_body(plo_ref, phi_ref, deg_ref, h_ref, wr1_ref, b1_ref, o_ref):
    agg = jnp.concatenate(
        [plo_ref[0] + plo_ref[1], phi_ref[0] + phi_ref[1]], axis=1)
    d = deg_ref[0, :, 0:1] + deg_ref[1, :, 0:1]
    agg = agg / jnp.maximum(d, 1.0)
    o_ref[...] = agg + jnp.dot(h_ref[...], wr1_ref[...],
                               preferred_element_type=jnp.float32) + b1_ref[...]


def _tc_final(p1lo, p1hi, degp, h, wr1, b1):
    return pl.pallas_call(
        _fin_body,
        grid=(GRID,),
        in_specs=[
            pl.BlockSpec((2, R, DH), lambda i: (0, i, 0)),
            pl.BlockSpec((2, R, DH), lambda i: (0, i, 0)),
            pl.BlockSpec((2, R, DEGW), lambda i: (0, i, 0)),
            pl.BlockSpec((R, D), lambda i: (i, 0)),
            pl.BlockSpec((D, D), lambda i: (0, 0)),
            pl.BlockSpec((1, D), lambda i: (0, 0)),
        ],
        out_specs=pl.BlockSpec((R, D), lambda i: (i, 0)),
        out_shape=jax.ShapeDtypeStruct((N_NODES, D), jnp.float32),
    )(p1lo, p1hi, degp, h, wr1, b1)


@jax.jit
def kernel(x, edge_index, Wl0, Wr0, b0, Wl1, Wr1, b1):
    ei = edge_index.astype(jnp.int32)
    pad = EP - N_EDGES
    src = jnp.concatenate([ei[0], jnp.zeros((pad,), jnp.int32)])
    dst = jnp.concatenate([ei[1], jnp.full((pad,), DUMMY_DST, jnp.int32)])
    src = src.reshape(CHUNKS_PER_W, NW, C).transpose(1, 0, 2)
    dst = dst.reshape(CHUNKS_PER_W, NW, C).transpose(1, 0, 2)
    b0r = b0.reshape(1, D)
    b1r = b1.reshape(1, D)

    y0lo, y0hi = _tc_matmul_split(x, Wl0)
    p0lo, p0hi, degp = _sc_aggregate_deg(y0lo, y0hi, src, dst)
    h, y1lo, y1hi = _tc_mid(p0lo, p0hi, degp, x, Wr0, b0r, Wl1)
    p1lo, p1hi = _sc_aggregate(y1lo, y1hi, src, dst)
    return _tc_final(p1lo, p1hi, degp, h, Wr1, b1r)


# speedup vs baseline: 2.3593x; 1.0798x over previous
import functools

import jax
import jax.numpy as jnp
from jax import lax
from jax.experimental import pallas as pl
from jax.experimental.pallas import tpu as pltpu
from jax.experimental.pallas import tpu_sc as plsc

N_NODES = 10000
N_EDGES = 320000
D = 128
DH = 64

C = 128
NW = 32
NP = 10240
EP_CHUNKS = 2528
EP = EP_CHUNKS * C
CHUNKS_PER_W = EP_CHUNKS // NW
ZCH_PER_SUB = NP // C // 16
DEGW = 16
DUMMY_DST = N_NODES + 8


def _sc_body(with_deg, *refs):
    if with_deg:
        (ylo_hbm, yhi_hbm, src_hbm, dst_hbm,
         outlo_hbm, outhi_hbm, deg_hbm,
         src_all, dst_all, rows_v, rows1_v, rows2_v, zbuf_v, ones_v,
         accum_s, deg_s, sem, sem1, sem2) = refs
    else:
        (ylo_hbm, yhi_hbm, src_hbm, dst_hbm,
         outlo_hbm, outhi_hbm,
         src_all, dst_all, rows_v, rows1_v, rows2_v, zbuf_v,
         accum_s, sem, sem1, sem2) = refs

    c = lax.axis_index("c")
    s = lax.axis_index("s")
    wid = s * 2 + c

    def zrow(r, _):
        def zcol(k, _):
            zbuf_v[r, pl.ds(k * 16, 16)] = jnp.zeros((16,), jnp.float32)
            return 0
        return lax.fori_loop(0, DH // 16, zcol, 0)
    lax.fori_loop(0, C, zrow, 0)

    if with_deg:
        def zo(r, _):
            ones_v[r, :] = jnp.zeros((DEGW,), jnp.float32)
            return 0
        lax.fori_loop(0, C, zo, 0)

    def zero_accum():
        for j in range(ZCH_PER_SUB):
            pltpu.sync_copy(zbuf_v, accum_s.at[pl.ds((s + 16 * j) * C, C)])

    pltpu.sync_copy(src_hbm.at[wid], src_all)
    pltpu.sync_copy(dst_hbm.at[wid], dst_all)

    zero_accum()
    if with_deg:
        for j in range(ZCH_PER_SUB):
            pltpu.sync_copy(ones_v, deg_s.at[pl.ds((s + 16 * j) * C, C)])

        def so(r, _):
            ones_v[r, :] = jnp.full((DEGW,), 1.0, jnp.float32)
            return 0
        lax.fori_loop(0, C, so, 0)

    plsc.subcore_barrier()

    def edge_pass(y_ref, add_deg):
        def scat(buf, j):
            pltpu.sync_copy(buf, accum_s.at[dst_all.at[j]], add=True)
            if add_deg:
                pltpu.sync_copy(ones_v, deg_s.at[dst_all.at[j]], add=True)

        def gath(j, buf, sm):
            pltpu.async_copy(y_ref.at[src_all.at[j]], buf, sm)

        def gwait(j, buf, sm):
            pltpu.make_async_copy(y_ref.at[src_all.at[j]], buf, sm).wait()

        gath(0, rows_v, sem)
        gath(1, rows1_v, sem1)
        gath(2, rows2_v, sem2)

        @pl.loop(0, CHUNKS_PER_W - 4, step=3)
        def chunk(j):
            gwait(j, rows_v, sem)
            scat(rows_v, j)
            gath(j + 3, rows_v, sem)
            gwait(j + 1, rows1_v, sem1)
            scat(rows1_v, j + 1)
            gath(j + 4, rows1_v, sem1)
            gwait(j + 2, rows2_v, sem2)
            scat(rows2_v, j + 2)
            gath(j + 5, rows2_v, sem2)

        gwait(75, rows_v, sem)
        scat(rows_v, 75)
        gath(78, rows_v, sem)
        gwait(76, rows1_v, sem1)
        scat(rows1_v, 76)
        gwait(77, rows2_v, sem2)
        scat(rows2_v, 77)
        gwait(78, rows_v, sem)
        scat(rows_v, 78)

    def writeback(dst_ref):
        for j in range(ZCH_PER_SUB):
            off = (s + 16 * j) * C
            pltpu.sync_copy(accum_s.at[pl.ds(off, C)],
                            dst_ref.at[c, pl.ds(off, C)])

    edge_pass(ylo_hbm, with_deg)
    plsc.subcore_barrier()
    writeback(outlo_hbm)
    if with_deg:
        for j in range(ZCH_PER_SUB):
            off = (s + 16 * j) * C
            pltpu.sync_copy(deg_s.at[pl.ds(off, C)],
                            deg_hbm.at[c, pl.ds(off, C)])
    zero_accum()
    plsc.subcore_barrier()

    edge_pass(yhi_hbm, False)
    plsc.subcore_barrier()
    writeback(outhi_hbm)


def _make_sc_aggregate(with_deg):
    mesh = plsc.VectorSubcoreMesh(core_axis_name="c", subcore_axis_name="s")
    out_type = [
        jax.ShapeDtypeStruct((2, NP, DH), jnp.float32),
        jax.ShapeDtypeStruct((2, NP, DH), jnp.float32),
    ]
    scratch = [
        pltpu.VMEM((CHUNKS_PER_W, C), jnp.int32),
        pltpu.VMEM((CHUNKS_PER_W, C), jnp.int32),
        pltpu.VMEM((C, DH), jnp.float32),
        pltpu.VMEM((C, DH), jnp.float32),
        pltpu.VMEM((C, DH), jnp.float32),
        pltpu.VMEM((C, DH), jnp.float32),
    ]
    if with_deg:
        out_type.append(jax.ShapeDtypeStruct((2, NP, DEGW), jnp.float32))
        scratch.append(pltpu.VMEM((C, DEGW), jnp.float32))
    scratch.append(pltpu.VMEM_SHARED((NP, DH), jnp.float32))
    if with_deg:
        scratch.append(pltpu.VMEM_SHARED((NP, DEGW), jnp.float32))
    scratch.append(pltpu.SemaphoreType.DMA)
    scratch.append(pltpu.SemaphoreType.DMA)
    scratch.append(pltpu.SemaphoreType.DMA)

    return pl.kernel(
        functools.partial(_sc_body, with_deg),
        out_type=tuple(out_type),
        mesh=mesh,
        scratch_types=tuple(scratch),
        compiler_params=pltpu.CompilerParams(use_tc_tiling_on_sc=False),
    )


_sc_aggregate_deg = _make_sc_aggregate(True)
_sc_aggregate = _make_sc_aggregate(False)


R = 400
GRID = N_NODES // R


def _mm_body(x_ref, w_ref, lo_ref, hi_ref):
    y = jnp.dot(x_ref[...], w_ref[...], preferred_element_type=jnp.float32)
    lo_ref[...] = y[:, :DH]
    hi_ref[...] = y[:, DH:]


def _tc_matmul_split(x, w):
    return pl.pallas_call(
        _mm_body,
        grid=(GRID,),
        in_specs=[
            pl.BlockSpec((R, D), lambda i: (i, 0)),
            pl.BlockSpec((D, D), lambda i: (0, 0)),
        ],
        out_specs=[
            pl.BlockSpec((R, DH), lambda i: (i, 0)),
            pl.BlockSpec((R, DH), lambda i: (i, 0)),
        ],
        out_shape=[
            jax.ShapeDtypeStruct((N_NODES, DH), jnp.float32),
            jax.ShapeDtypeStruct((N_NODES, DH), jnp.float32),
        ],
    )(x, w)


def _mid_body(plo_ref, phi_ref, deg_ref, x_ref, wr0_ref, b0_ref, wl1_ref,
              h_ref, y1lo_ref, y1hi_ref):
    agg = jnp.concatenate(
        [plo_ref[0] + plo_ref[1], phi_ref[0] + phi_ref[1]], axis=1)
    d = deg_ref[0, :, 0:1] + deg_ref[1, :, 0:1]
    agg = agg / jnp.maximum(d, 1.0)
    h = agg + jnp.dot(x_ref[...], wr0_ref[...],
                      preferred_element_type=jnp.float32) + b0_ref[...]
    h = jnp.maximum(h, 0.0)
    h_ref[...] = h
    y1 = jnp.dot(h, wl1_ref[...], preferred_element_type=jnp.float32)
    y1lo_ref[...] = y1[:, :DH]
    y1hi_ref[...] = y1[:, DH:]


def _tc_mid(p0lo, p0hi, degp, x, wr0, b0, wl1):
    return pl.pallas_call(
        _mid_body,
        grid=(GRID,),
        in_specs=[
            pl.BlockSpec((2, R, DH), lambda i: (0, i, 0)),
            pl.BlockSpec((2, R, DH), lambda i: (0, i, 0)),
            pl.BlockSpec((2, R, DEGW), lambda i: (0, i, 0)),
            pl.BlockSpec((R, D), lambda i: (i, 0)),
            pl.BlockSpec((D, D), lambda i: (0, 0)),
            pl.BlockSpec((1, D), lambda i: (0, 0)),
            pl.BlockSpec((D, D), lambda i: (0, 0)),
        ],
        out_specs=[
            pl.BlockSpec((R, D), lambda i: (i, 0)),
            pl.BlockSpec((R, DH), lambda i: (i, 0)),
            pl.BlockSpec((R, DH), lambda i: (i, 0)),
        ],
        out_shape=[
            jax.ShapeDtypeStruct((N_NODES, D), jnp.float32),
            jax.ShapeDtypeStruct((N_NODES, DH), jnp.float32),
            jax.ShapeDtypeStruct((N_NODES, DH), jnp.float32),
        ],
    )(p0lo, p0hi, degp, x, wr0, b0, wl1)


def _fin_body(plo_ref, phi_ref, deg_ref, h_ref, wr1_ref, b1_ref, o_ref):
    agg = jnp.concatenate(
        [plo_ref[0] + plo_ref[1], phi_ref[0] + phi_ref[1]], axis=1)
    d = deg_ref[0, :, 0:1] + deg_ref[1, :, 0:1]
    agg = agg / jnp.maximum(d, 1.0)
    o_ref[...] = agg + jnp.dot(h_ref[...], wr1_ref[...],
                               preferred_element_type=jnp.float32) + b1_ref[...]


def _tc_final(p1lo, p1hi, degp, h, wr1, b1):
    return pl.pallas_call(
        _fin_body,
        grid=(GRID,),
        in_specs=[
            pl.BlockSpec((2, R, DH), lambda i: (0, i, 0)),
            pl.BlockSpec((2, R, DH), lambda i: (0, i, 0)),
            pl.BlockSpec((2, R, DEGW), lambda i: (0, i, 0)),
            pl.BlockSpec((R, D), lambda i: (i, 0)),
            pl.BlockSpec((D, D), lambda i: (0, 0)),
            pl.BlockSpec((1, D), lambda i: (0, 0)),
        ],
        out_specs=pl.BlockSpec((R, D), lambda i: (i, 0)),
        out_shape=jax.ShapeDtypeStruct((N_NODES, D), jnp.float32),
    )(p1lo, p1hi, degp, h, wr1, b1)


@jax.jit
def kernel(x, edge_index, Wl0, Wr0, b0, Wl1, Wr1, b1):
    ei = edge_index.astype(jnp.int32)
    pad = EP - N_EDGES
    src = jnp.concatenate([ei[0], jnp.zeros((pad,), jnp.int32)])
    dst = jnp.concatenate([ei[1], jnp.full((pad,), DUMMY_DST, jnp.int32)])
    src = src.reshape(CHUNKS_PER_W, NW, C).transpose(1, 0, 2)
    dst = dst.reshape(CHUNKS_PER_W, NW, C).transpose(1, 0, 2)
    b0r = b0.reshape(1, D)
    b1r = b1.reshape(1, D)

    y0lo, y0hi = _tc_matmul_split(x, Wl0)
    p0lo, p0hi, degp = _sc_aggregate_deg(y0lo, y0hi, src, dst)
    h, y1lo, y1hi = _tc_mid(p0lo, p0hi, degp, x, Wr0, b0r, Wl1)
    p1lo, p1hi = _sc_aggregate(y1lo, y1hi, src, dst)
    return _tc_final(p1lo, p1hi, degp, h, Wr1, b1r)
